# baseline (device time: 712298 ns/iter reference)
import jax
import jax.numpy as jnp
from jax import lax
from jax.experimental import pallas as pl
from jax.experimental.pallas import tpu as pltpu

N_DEV = 8


def kernel(O, Wo):
    B, S, H, D = O.shape
    K = H * D
    n_out = Wo.shape[1]
    s_per = S // N_DEV

    Ob = O.reshape(B, S, K).astype(jnp.bfloat16)
    Wb = Wo.astype(jnp.bfloat16)

    def body(o_ref, w_ref, out_ref, comm_ref, send_sems, recv_sems):
        my = lax.axis_index("i")
        left = (my + N_DEV - 1) % N_DEV
        right = (my + 1) % N_DEV

        barrier_sem = pltpu.get_barrier_semaphore()
        for nbr in (left, right):
            pl.semaphore_signal(barrier_sem, inc=1, device_id=(nbr,),
                                device_id_type=pl.DeviceIdType.MESH)
        pl.semaphore_wait(barrier_sem, 2)

        def partial(c, b):
            x = o_ref[b, pl.ds(c * s_per, s_per), :]
            return jnp.dot(x, w_ref[...], preferred_element_type=jnp.float32)

        c0 = (my + N_DEV - 1) % N_DEV
        for b in range(B):
            comm_ref[0, b] = partial(c0, b).astype(jnp.bfloat16)

        for t in range(N_DEV - 1):
            send_slot = t % 2
            recv_slot = (t + 1) % 2
            rdma = pltpu.make_async_remote_copy(
                src_ref=comm_ref.at[send_slot],
                dst_ref=comm_ref.at[recv_slot],
                send_sem=send_sems.at[t],
                recv_sem=recv_sems.at[t],
                device_id=(right,),
                device_id_type=pl.DeviceIdType.MESH,
            )
            rdma.start()
            c = (my + 2 * N_DEV - t - 2) % N_DEV
            for b in range(B):
                out_ref[b] = partial(c, b)
            rdma.wait()
            if t < N_DEV - 2:
                for b in range(B):
                    comm_ref[recv_slot, b] = (
                        comm_ref[recv_slot, b].astype(jnp.float32) + out_ref[b]
                    ).astype(jnp.bfloat16)
            else:
                for b in range(B):
                    out_ref[b] = out_ref[b] + comm_ref[recv_slot, b].astype(
                        jnp.float32)

    return pl.pallas_call(
        body,
        out_shape=jax.ShapeDtypeStruct((B, s_per, n_out), jnp.float32),
        in_specs=[pl.BlockSpec(memory_space=pltpu.VMEM),
                  pl.BlockSpec(memory_space=pltpu.VMEM)],
        out_specs=pl.BlockSpec(memory_space=pltpu.VMEM),
        scratch_shapes=[
            pltpu.VMEM((2, B, s_per, n_out), jnp.bfloat16),
            pltpu.SemaphoreType.DMA((N_DEV - 1,)),
            pltpu.SemaphoreType.DMA((N_DEV - 1,)),
        ],
        compiler_params=pltpu.CompilerParams(
            collective_id=0,
            vmem_limit_bytes=100 * 1024 * 1024,
        ),
    )(Ob, Wb)


# device time: 707119 ns/iter; 1.0073x vs baseline; 1.0073x over previous
import jax
import jax.numpy as jnp
from jax import lax
from jax.experimental import pallas as pl
from jax.experimental.pallas import tpu as pltpu

N_DEV = 8


def kernel(O, Wo):
    B, S, H, D = O.shape
    K = H * D
    n_out = Wo.shape[1]
    s_per = S // N_DEV

    Ob = O.reshape(B, S, K).astype(jnp.bfloat16)
    Wb = Wo.astype(jnp.bfloat16)

    n_split = 2
    n_half = n_out // n_split

    def body(o_ref, w_ref, out_ref, comm_ref, stage_ref, send_sems, recv_sems):
        my = lax.axis_index("i")
        left = (my + N_DEV - 1) % N_DEV
        right = (my + 1) % N_DEV

        barrier_sem = pltpu.get_barrier_semaphore()
        for nbr in (left, right):
            pl.semaphore_signal(barrier_sem, inc=1, device_id=(nbr,),
                                device_id_type=pl.DeviceIdType.MESH)
        pl.semaphore_wait(barrier_sem, 2)

        def partial(c, b):
            x = o_ref[b, pl.ds(c * s_per, s_per), :]
            return jnp.dot(x, w_ref[...], preferred_element_type=jnp.float32)

        c0 = (my + N_DEV - 1) % N_DEV
        for b in range(B):
            comm_ref[0, b] = partial(c0, b).astype(jnp.bfloat16)

        for t in range(N_DEV - 1):
            send_slot = t % 2
            recv_slot = (t + 1) % 2
            rdmas = []
            for h in range(n_split):
                sl = pl.ds(h * n_half, n_half)
                rdma = pltpu.make_async_remote_copy(
                    src_ref=comm_ref.at[send_slot, :, :, sl],
                    dst_ref=comm_ref.at[recv_slot, :, :, sl],
                    send_sem=send_sems.at[t, h],
                    recv_sem=recv_sems.at[t, h],
                    device_id=(right,),
                    device_id_type=pl.DeviceIdType.MESH,
                )
                rdma.start()
                rdmas.append(rdma)
            c = (my + 2 * N_DEV - t - 2) % N_DEV
            for b in range(B):
                stage_ref[b] = partial(c, b).astype(jnp.bfloat16)
            for h in range(n_split):
                sl = pl.ds(h * n_half, n_half)
                rdmas[h].wait()
                if t < N_DEV - 2:
                    comm_ref[recv_slot, :, :, sl] = (
                        comm_ref[recv_slot, :, :, sl].astype(jnp.float32)
                        + stage_ref[:, :, sl].astype(jnp.float32)
                    ).astype(jnp.bfloat16)
                else:
                    out_ref[:, :, sl] = (
                        comm_ref[recv_slot, :, :, sl].astype(jnp.float32)
                        + stage_ref[:, :, sl].astype(jnp.float32)
                    )

    return pl.pallas_call(
        body,
        out_shape=jax.ShapeDtypeStruct((B, s_per, n_out), jnp.float32),
        in_specs=[pl.BlockSpec(memory_space=pltpu.VMEM),
                  pl.BlockSpec(memory_space=pltpu.VMEM)],
        out_specs=pl.BlockSpec(memory_space=pltpu.VMEM),
        scratch_shapes=[
            pltpu.VMEM((2, B, s_per, n_out), jnp.bfloat16),
            pltpu.VMEM((B, s_per, n_out), jnp.bfloat16),
            pltpu.SemaphoreType.DMA((N_DEV - 1, n_split)),
            pltpu.SemaphoreType.DMA((N_DEV - 1, n_split)),
        ],
        compiler_params=pltpu.CompilerParams(
            collective_id=0,
            vmem_limit_bytes=100 * 1024 * 1024,
        ),
    )(Ob, Wb)


# device time: 681971 ns/iter; 1.0445x vs baseline; 1.0369x over previous
import jax
import jax.numpy as jnp
from jax import lax
from jax.experimental import pallas as pl
from jax.experimental.pallas import tpu as pltpu

N_DEV = 8


def kernel(O, Wo):
    B, S, H, D = O.shape
    K = H * D
    n_out = Wo.shape[1]
    s_per = S // N_DEV

    Ob = O.reshape(B, S, K).astype(jnp.bfloat16)
    Wb = Wo.astype(jnp.bfloat16)

    n_split = 2
    n_half = n_out // n_split

    def body(o_ref, w_ref, out_ref, comm_ref, stage_ref, send_sems, recv_sems):
        my = lax.axis_index("i")
        left = (my + N_DEV - 1) % N_DEV
        right = (my + 1) % N_DEV

        barrier_sem = pltpu.get_barrier_semaphore()
        for nbr in (left, right):
            pl.semaphore_signal(barrier_sem, inc=1, device_id=(nbr,),
                                device_id_type=pl.DeviceIdType.MESH)
        pl.semaphore_wait(barrier_sem, 2)

        def partial(c, b):
            x = o_ref[b, pl.ds(c * s_per, s_per), :]
            return jnp.dot(x, w_ref[...], preferred_element_type=jnp.float32)

        def partial_half(c, b, h):
            x = o_ref[b, pl.ds(c * s_per, s_per), :]
            w = w_ref[:, pl.ds(h * n_half, n_half)]
            return jnp.dot(x, w, preferred_element_type=jnp.float32)

        def make_rdma(t, h, send_slot, recv_slot):
            sl = pl.ds(h * n_half, n_half)
            return pltpu.make_async_remote_copy(
                src_ref=comm_ref.at[send_slot, :, :, sl],
                dst_ref=comm_ref.at[recv_slot, :, :, sl],
                send_sem=send_sems.at[t, h],
                recv_sem=recv_sems.at[t, h],
                device_id=(right,),
                device_id_type=pl.DeviceIdType.MESH,
            )

        def compute_stage(t):
            c = (my + 2 * N_DEV - t - 2) % N_DEV
            for b in range(B):
                stage_ref[b] = partial(c, b).astype(jnp.bfloat16)

        c0 = (my + N_DEV - 1) % N_DEV
        rdmas = [None] * n_split
        for h in range(n_split):
            sl = pl.ds(h * n_half, n_half)
            for b in range(B):
                comm_ref[0, b, :, sl] = partial_half(c0, b, h).astype(
                    jnp.bfloat16)
            rdmas[h] = make_rdma(0, h, 0, 1)
            rdmas[h].start()
        compute_stage(0)

        for t in range(N_DEV - 1):
            send_slot = t % 2
            recv_slot = (t + 1) % 2
            for h in range(n_split):
                sl = pl.ds(h * n_half, n_half)
                rdmas[h].wait()
                if t < N_DEV - 2:
                    comm_ref[recv_slot, :, :, sl] = (
                        comm_ref[recv_slot, :, :, sl].astype(jnp.float32)
                        + stage_ref[:, :, sl].astype(jnp.float32)
                    ).astype(jnp.bfloat16)
                    rdmas[h] = make_rdma(t + 1, h, recv_slot, send_slot)
                    rdmas[h].start()
                else:
                    out_ref[:, :, sl] = (
                        comm_ref[recv_slot, :, :, sl].astype(jnp.float32)
                        + stage_ref[:, :, sl].astype(jnp.float32)
                    )
            if t < N_DEV - 2:
                compute_stage(t + 1)

    return pl.pallas_call(
        body,
        out_shape=jax.ShapeDtypeStruct((B, s_per, n_out), jnp.float32),
        in_specs=[pl.BlockSpec(memory_space=pltpu.VMEM),
                  pl.BlockSpec(memory_space=pltpu.VMEM)],
        out_specs=pl.BlockSpec(memory_space=pltpu.VMEM),
        scratch_shapes=[
            pltpu.VMEM((2, B, s_per, n_out), jnp.bfloat16),
            pltpu.VMEM((B, s_per, n_out), jnp.bfloat16),
            pltpu.SemaphoreType.DMA((N_DEV - 1, n_split)),
            pltpu.SemaphoreType.DMA((N_DEV - 1, n_split)),
        ],
        compiler_params=pltpu.CompilerParams(
            collective_id=0,
            vmem_limit_bytes=100 * 1024 * 1024,
        ),
    )(Ob, Wb)


# device time: 680584 ns/iter; 1.0466x vs baseline; 1.0020x over previous
import jax
import jax.numpy as jnp
from jax import lax
from jax.experimental import pallas as pl
from jax.experimental.pallas import tpu as pltpu

N_DEV = 8


def kernel(O, Wo):
    B, S, H, D = O.shape
    K = H * D
    n_out = Wo.shape[1]
    s_per = S // N_DEV

    Ob = jax.lax.optimization_barrier(O.astype(jnp.bfloat16))
    Ob = Ob.reshape(B, S, K)
    Wb = Wo.astype(jnp.bfloat16)

    n_split = 4
    n_half = n_out // n_split

    def body(o_ref, w_ref, out_ref, comm_ref, stage_ref, send_sems, recv_sems):
        my = lax.axis_index("i")
        left = (my + N_DEV - 1) % N_DEV
        right = (my + 1) % N_DEV

        barrier_sem = pltpu.get_barrier_semaphore()
        for nbr in (left, right):
            pl.semaphore_signal(barrier_sem, inc=1, device_id=(nbr,),
                                device_id_type=pl.DeviceIdType.MESH)
        pl.semaphore_wait(barrier_sem, 2)

        def partial(c, b):
            x = o_ref[b, pl.ds(c * s_per, s_per), :]
            return jnp.dot(x, w_ref[...], preferred_element_type=jnp.float32)

        def partial_half(c, b, h):
            x = o_ref[b, pl.ds(c * s_per, s_per), :]
            w = w_ref[:, pl.ds(h * n_half, n_half)]
            return jnp.dot(x, w, preferred_element_type=jnp.float32)

        def make_rdma(t, h, send_slot, recv_slot):
            sl = pl.ds(h * n_half, n_half)
            return pltpu.make_async_remote_copy(
                src_ref=comm_ref.at[send_slot, :, :, sl],
                dst_ref=comm_ref.at[recv_slot, :, :, sl],
                send_sem=send_sems.at[t, h],
                recv_sem=recv_sems.at[t, h],
                device_id=(right,),
                device_id_type=pl.DeviceIdType.MESH,
            )

        def compute_stage(t):
            c = (my + 2 * N_DEV - t - 2) % N_DEV
            for b in range(B):
                stage_ref[b] = partial(c, b).astype(jnp.bfloat16)

        c0 = (my + N_DEV - 1) % N_DEV
        rdmas = [None] * n_split
        for h in range(n_split):
            sl = pl.ds(h * n_half, n_half)
            for b in range(B):
                comm_ref[0, b, :, sl] = partial_half(c0, b, h).astype(
                    jnp.bfloat16)
            rdmas[h] = make_rdma(0, h, 0, 1)
            rdmas[h].start()
        compute_stage(0)

        for t in range(N_DEV - 1):
            send_slot = t % 2
            recv_slot = (t + 1) % 2
            for h in range(n_split):
                sl = pl.ds(h * n_half, n_half)
                rdmas[h].wait()
                if t < N_DEV - 2:
                    comm_ref[recv_slot, :, :, sl] = (
                        comm_ref[recv_slot, :, :, sl].astype(jnp.float32)
                        + stage_ref[:, :, sl].astype(jnp.float32)
                    ).astype(jnp.bfloat16)
                    rdmas[h] = make_rdma(t + 1, h, recv_slot, send_slot)
                    rdmas[h].start()
                else:
                    out_ref[:, :, sl] = (
                        comm_ref[recv_slot, :, :, sl].astype(jnp.float32)
                        + stage_ref[:, :, sl].astype(jnp.float32)
                    )
            if t < N_DEV - 2:
                compute_stage(t + 1)

    return pl.pallas_call(
        body,
        out_shape=jax.ShapeDtypeStruct((B, s_per, n_out), jnp.float32),
        in_specs=[pl.BlockSpec(memory_space=pltpu.VMEM),
                  pl.BlockSpec(memory_space=pltpu.VMEM)],
        out_specs=pl.BlockSpec(memory_space=pltpu.VMEM),
        scratch_shapes=[
            pltpu.VMEM((2, B, s_per, n_out), jnp.bfloat16),
            pltpu.VMEM((B, s_per, n_out), jnp.bfloat16),
            pltpu.SemaphoreType.DMA((N_DEV - 1, n_split)),
            pltpu.SemaphoreType.DMA((N_DEV - 1, n_split)),
        ],
        compiler_params=pltpu.CompilerParams(
            collective_id=0,
            vmem_limit_bytes=100 * 1024 * 1024,
        ),
    )(Ob, Wb)
